# Initial kernel scaffold; baseline (speedup 1.0000x reference)
import jax
import jax.numpy as jnp
from jax.experimental import pallas as pl


def kernel(source_node, target_node, edge_attr, distance, W1, b1, W2, b2, Wc1, bc1, Wc2, bc2, gn_weight, gn_bias, gn_mean_scale, edge_index, target_batch):
    raise NotImplementedError("write your pallas kernel here")


# speedup vs baseline: 1.0678x; 1.0678x over previous
import functools

import jax
import jax.numpy as jnp
from jax.experimental import pallas as pl
from jax.experimental.pallas import tpu as pltpu

N = 10000
E = 320000
D = 128
H = 128
G = 16

EBLK = 2000


def _precompute_body(src_ref, tgt_ref, ws_ref, wt_ref, b1_ref, a_ref, b_ref):
    a_ref[...] = jnp.dot(src_ref[...], ws_ref[...],
                         preferred_element_type=jnp.float32)
    b_ref[...] = jnp.dot(tgt_ref[...], wt_ref[...],
                         preferred_element_type=jnp.float32) + b1_ref[...]


def _edge_mlp_body(ga_ref, gb_ref, dist_ref, w1d_ref, w2t_ref, b2_ref, out_ref):
    pre = ga_ref[...] + gb_ref[...] + dist_ref[...] * w1d_ref[...]
    h = pre * jax.nn.sigmoid(pre)
    h2 = jnp.dot(h, w2t_ref[...], preferred_element_type=jnp.float32) + b2_ref[...]
    out_ref[...] = h2 * jax.nn.sigmoid(h2)


def _combine_norm_body(tgt_ref, aggr_ref, wc1t_t_ref, wc1t_a_ref, bc1_ref,
                       wc2t_ref, bc2_ref, gnw_ref, gnb_ref, gms_ref,
                       tb_col_ref, tb_row_ref, y_ref):
    tgt = tgt_ref[...]
    c1 = (jnp.dot(tgt, wc1t_t_ref[...], preferred_element_type=jnp.float32)
          + jnp.dot(aggr_ref[...], wc1t_a_ref[...],
                    preferred_element_type=jnp.float32)
          + bc1_ref[...])
    c1 = c1 * jax.nn.sigmoid(c1)
    c = jnp.dot(c1, wc2t_ref[...], preferred_element_type=jnp.float32) + bc2_ref[...]
    x = tgt + c

    iota_col = jax.lax.broadcasted_iota(jnp.int32, (N, G), 1)
    onehot = (iota_col == tb_col_ref[...]).astype(jnp.float32)
    iota_row = jax.lax.broadcasted_iota(jnp.int32, (G, N), 0)
    onehot_t = (iota_row == tb_row_ref[...]).astype(jnp.float32)

    counts = jnp.maximum(jnp.sum(onehot_t, axis=1, keepdims=True), 1.0)
    mean = jnp.dot(onehot_t, x, preferred_element_type=jnp.float32) / counts
    out = x - gms_ref[...] * jnp.dot(onehot, mean,
                                     preferred_element_type=jnp.float32)
    var = jnp.dot(onehot_t, out * out,
                  preferred_element_type=jnp.float32) / counts
    rstd = jax.lax.rsqrt(var + 1e-5)
    y_ref[...] = (gnw_ref[...] * out
                  * jnp.dot(onehot, rstd, preferred_element_type=jnp.float32)
                  + gnb_ref[...])


def kernel(source_node, target_node, edge_attr, distance, W1, b1, W2, b2,
           Wc1, bc1, Wc2, bc2, gn_weight, gn_bias, gn_mean_scale,
           edge_index, target_batch):
    del edge_attr
    i_src = edge_index[0]
    i_tgt = edge_index[1]

    W1sT = W1[:, :D].T
    W1tT = W1[:, D:2 * D].T
    w1d = W1[:, 2 * D].reshape(1, H)
    b1r = b1.reshape(1, H)
    W2T = W2.T
    b2r = b2.reshape(1, H)
    Wc1T_t = Wc1[:, :D].T
    Wc1T_a = Wc1[:, D:].T
    bc1r = bc1.reshape(1, H)
    Wc2T = Wc2.T
    bc2r = bc2.reshape(1, H)
    gnw = gn_weight.reshape(1, H)
    gnb = gn_bias.reshape(1, H)
    gms = gn_mean_scale.reshape(1, H)
    tb_col = target_batch.reshape(N, 1)
    tb_row = target_batch.reshape(1, N)

    A, B = pl.pallas_call(
        _precompute_body,
        out_shape=[jax.ShapeDtypeStruct((N, H), jnp.float32),
                   jax.ShapeDtypeStruct((N, H), jnp.float32)],
    )(source_node, target_node, W1sT, W1tT, b1r)

    Ga = jnp.take(A, i_src, axis=0)
    Gb = jnp.take(B, i_tgt, axis=0)

    grid = E // EBLK
    H2 = pl.pallas_call(
        _edge_mlp_body,
        grid=(grid,),
        in_specs=[
            pl.BlockSpec((EBLK, H), lambda i: (i, 0)),
            pl.BlockSpec((EBLK, H), lambda i: (i, 0)),
            pl.BlockSpec((EBLK, 1), lambda i: (i, 0)),
            pl.BlockSpec((1, H), lambda i: (0, 0)),
            pl.BlockSpec((H, H), lambda i: (0, 0)),
            pl.BlockSpec((1, H), lambda i: (0, 0)),
        ],
        out_specs=pl.BlockSpec((EBLK, H), lambda i: (i, 0)),
        out_shape=jax.ShapeDtypeStruct((E, H), jnp.float32),
    )(Ga, Gb, distance, w1d, W2T, b2r)

    aggr = jax.ops.segment_sum(H2, i_tgt, num_segments=N)

    y = pl.pallas_call(
        _combine_norm_body,
        out_shape=jax.ShapeDtypeStruct((N, H), jnp.float32),
    )(target_node, aggr, Wc1T_t, Wc1T_a, bc1r, Wc2T, bc2r,
      gnw, gnb, gms, tb_col, tb_row)

    return y


# speedup vs baseline: 2.1609x; 2.0237x over previous
import functools

import jax
import jax.numpy as jnp
from jax import lax
from jax.experimental import pallas as pl
from jax.experimental.pallas import tpu as pltpu
from jax.experimental.pallas import tpu_sc as plsc

N = 10000
E = 320000
D = 128
H = 128
G = 16

EBLK = 2000

NC = 2
NS = 16
NW = NC * NS
EW = E // NW
CH = 80
NCH = EW // CH


def _sc_gather(A, B, i_src, i_tgt):
    mesh = plsc.VectorSubcoreMesh(core_axis_name="c", subcore_axis_name="s")

    @functools.partial(
        pl.kernel, mesh=mesh,
        out_type=[jax.ShapeDtypeStruct((E, D), jnp.float32),
                  jax.ShapeDtypeStruct((E, D), jnp.float32)],
        scratch_types=[
            pltpu.VMEM((CH,), jnp.int32),
            pltpu.VMEM((CH,), jnp.int32),
            pltpu.VMEM((CH, D), jnp.float32),
            pltpu.VMEM((CH, D), jnp.float32),
            pltpu.SemaphoreType.DMA,
        ],
    )
    def gk(a_hbm, b_hbm, is_hbm, it_hbm, ga_hbm, gb_hbm,
           idx_s, idx_t, rows_a, rows_b, sem):
        wid = lax.axis_index("s") * NC + lax.axis_index("c")
        base = wid * EW

        def body(k, carry):
            off = base + k * CH
            pltpu.sync_copy(is_hbm.at[pl.ds(off, CH)], idx_s)
            pltpu.sync_copy(it_hbm.at[pl.ds(off, CH)], idx_t)
            ca = pltpu.async_copy(a_hbm.at[idx_s], rows_a, sem)
            cb = pltpu.async_copy(b_hbm.at[idx_t], rows_b, sem)
            ca.wait()
            cb.wait()
            pltpu.sync_copy(rows_a, ga_hbm.at[pl.ds(off, CH)])
            pltpu.sync_copy(rows_b, gb_hbm.at[pl.ds(off, CH)])
            return carry

        lax.fori_loop(0, NCH, body, 0)

    return gk(A, B, i_src, i_tgt)


def _precompute_body(src_ref, tgt_ref, ws_ref, wt_ref, b1_ref, a_ref, b_ref):
    a_ref[...] = jnp.dot(src_ref[...], ws_ref[...],
                         preferred_element_type=jnp.float32)
    b_ref[...] = jnp.dot(tgt_ref[...], wt_ref[...],
                         preferred_element_type=jnp.float32) + b1_ref[...]


def _edge_mlp_body(ga_ref, gb_ref, dist_ref, w1d_ref, w2t_ref, b2_ref, out_ref):
    pre = ga_ref[...] + gb_ref[...] + dist_ref[...] * w1d_ref[...]
    h = pre * jax.nn.sigmoid(pre)
    h2 = jnp.dot(h, w2t_ref[...], preferred_element_type=jnp.float32) + b2_ref[...]
    out_ref[...] = h2 * jax.nn.sigmoid(h2)


def _combine_norm_body(tgt_ref, aggr_ref, wc1t_t_ref, wc1t_a_ref, bc1_ref,
                       wc2t_ref, bc2_ref, gnw_ref, gnb_ref, gms_ref,
                       tb_col_ref, tb_row_ref, y_ref):
    tgt = tgt_ref[...]
    c1 = (jnp.dot(tgt, wc1t_t_ref[...], preferred_element_type=jnp.float32)
          + jnp.dot(aggr_ref[...], wc1t_a_ref[...],
                    preferred_element_type=jnp.float32)
          + bc1_ref[...])
    c1 = c1 * jax.nn.sigmoid(c1)
    c = jnp.dot(c1, wc2t_ref[...], preferred_element_type=jnp.float32) + bc2_ref[...]
    x = tgt + c

    iota_col = jax.lax.broadcasted_iota(jnp.int32, (N, G), 1)
    onehot = (iota_col == tb_col_ref[...]).astype(jnp.float32)
    iota_row = jax.lax.broadcasted_iota(jnp.int32, (G, N), 0)
    onehot_t = (iota_row == tb_row_ref[...]).astype(jnp.float32)

    counts = jnp.maximum(jnp.sum(onehot_t, axis=1, keepdims=True), 1.0)
    mean = jnp.dot(onehot_t, x, preferred_element_type=jnp.float32) / counts
    out = x - gms_ref[...] * jnp.dot(onehot, mean,
                                     preferred_element_type=jnp.float32)
    var = jnp.dot(onehot_t, out * out,
                  preferred_element_type=jnp.float32) / counts
    rstd = jax.lax.rsqrt(var + 1e-5)
    y_ref[...] = (gnw_ref[...] * out
                  * jnp.dot(onehot, rstd, preferred_element_type=jnp.float32)
                  + gnb_ref[...])


def kernel(source_node, target_node, edge_attr, distance, W1, b1, W2, b2,
           Wc1, bc1, Wc2, bc2, gn_weight, gn_bias, gn_mean_scale,
           edge_index, target_batch):
    del edge_attr
    i_src = edge_index[0]
    i_tgt = edge_index[1]

    W1sT = W1[:, :D].T
    W1tT = W1[:, D:2 * D].T
    w1d = W1[:, 2 * D].reshape(1, H)
    b1r = b1.reshape(1, H)
    W2T = W2.T
    b2r = b2.reshape(1, H)
    Wc1T_t = Wc1[:, :D].T
    Wc1T_a = Wc1[:, D:].T
    bc1r = bc1.reshape(1, H)
    Wc2T = Wc2.T
    bc2r = bc2.reshape(1, H)
    gnw = gn_weight.reshape(1, H)
    gnb = gn_bias.reshape(1, H)
    gms = gn_mean_scale.reshape(1, H)
    tb_col = target_batch.reshape(N, 1)
    tb_row = target_batch.reshape(1, N)

    A, B = pl.pallas_call(
        _precompute_body,
        out_shape=[jax.ShapeDtypeStruct((N, H), jnp.float32),
                   jax.ShapeDtypeStruct((N, H), jnp.float32)],
    )(source_node, target_node, W1sT, W1tT, b1r)

    Ga, Gb = _sc_gather(A, B, i_src, i_tgt)

    grid = E // EBLK
    H2 = pl.pallas_call(
        _edge_mlp_body,
        grid=(grid,),
        in_specs=[
            pl.BlockSpec((EBLK, H), lambda i: (i, 0)),
            pl.BlockSpec((EBLK, H), lambda i: (i, 0)),
            pl.BlockSpec((EBLK, 1), lambda i: (i, 0)),
            pl.BlockSpec((1, H), lambda i: (0, 0)),
            pl.BlockSpec((H, H), lambda i: (0, 0)),
            pl.BlockSpec((1, H), lambda i: (0, 0)),
        ],
        out_specs=pl.BlockSpec((EBLK, H), lambda i: (i, 0)),
        out_shape=jax.ShapeDtypeStruct((E, H), jnp.float32),
    )(Ga, Gb, distance, w1d, W2T, b2r)

    aggr = jax.ops.segment_sum(H2, i_tgt, num_segments=N)

    y = pl.pallas_call(
        _combine_norm_body,
        out_shape=jax.ShapeDtypeStruct((N, H), jnp.float32),
    )(target_node, aggr, Wc1T_t, Wc1T_a, bc1r, Wc2T, bc2r,
      gnw, gnb, gms, tb_col, tb_row)

    return y


# speedup vs baseline: 3.0961x; 1.4328x over previous
import functools

import jax
import jax.numpy as jnp
from jax import lax
from jax.experimental import pallas as pl
from jax.experimental.pallas import tpu as pltpu
from jax.experimental.pallas import tpu_sc as plsc

N = 10000
E = 320000
D = 128
H = 128
G = 16

EBLK = 2000

NC = 2
NS = 16
NW = NC * NS
EW = E // NW
CH = 80
NCH = EW // CH


def _sc_gather(A, B, i_src, i_tgt):
    mesh = plsc.VectorSubcoreMesh(core_axis_name="c", subcore_axis_name="s")

    @functools.partial(
        pl.kernel, mesh=mesh,
        out_type=[jax.ShapeDtypeStruct((E, D), jnp.float32),
                  jax.ShapeDtypeStruct((E, D), jnp.float32)],
        scratch_types=[
            pltpu.VMEM((CH,), jnp.int32),
            pltpu.VMEM((CH,), jnp.int32),
            pltpu.VMEM((CH, D), jnp.float32),
            pltpu.VMEM((CH, D), jnp.float32),
            pltpu.SemaphoreType.DMA,
        ],
    )
    def gk(a_hbm, b_hbm, is_hbm, it_hbm, ga_hbm, gb_hbm,
           idx_s, idx_t, rows_a, rows_b, sem):
        wid = lax.axis_index("s") * NC + lax.axis_index("c")
        base = wid * EW

        def body(k, carry):
            off = base + k * CH
            pltpu.sync_copy(is_hbm.at[pl.ds(off, CH)], idx_s)
            pltpu.sync_copy(it_hbm.at[pl.ds(off, CH)], idx_t)
            ca = pltpu.async_copy(a_hbm.at[idx_s], rows_a, sem)
            cb = pltpu.async_copy(b_hbm.at[idx_t], rows_b, sem)
            ca.wait()
            cb.wait()
            pltpu.sync_copy(rows_a, ga_hbm.at[pl.ds(off, CH)])
            pltpu.sync_copy(rows_b, gb_hbm.at[pl.ds(off, CH)])
            return carry

        lax.fori_loop(0, NCH, body, 0)

    return gk(A, B, i_src, i_tgt)


NRA = 632
NPAD = NRA * NS


def _sc_scatter_add(H2, i_tgt, zeros_nd):
    mesh = plsc.VectorSubcoreMesh(core_axis_name="c", subcore_axis_name="s")

    @functools.partial(
        pl.kernel, mesh=mesh,
        out_type=jax.ShapeDtypeStruct((NC, NPAD, D), jnp.float32),
        scratch_types=[
            pltpu.VMEM((CH,), jnp.int32),
            pltpu.VMEM((CH, D), jnp.float32),
            pltpu.VMEM_SHARED((NPAD, D), jnp.float32),
        ],
    )
    def sk(h2_hbm, it_hbm, z_hbm, out_hbm, idx_t, rows, acc):
        c = lax.axis_index("c")
        s = lax.axis_index("s")
        pltpu.sync_copy(z_hbm, acc.at[pl.ds(s * NRA, NRA)])
        plsc.subcore_barrier()

        base = (s * NC + c) * EW

        def body(k, carry):
            off = base + k * CH
            pltpu.sync_copy(it_hbm.at[pl.ds(off, CH)], idx_t)
            pltpu.sync_copy(h2_hbm.at[pl.ds(off, CH)], rows)
            pltpu.sync_copy(rows, acc.at[idx_t], add=True)
            return carry

        lax.fori_loop(0, NCH, body, 0)
        plsc.subcore_barrier()
        pltpu.sync_copy(acc.at[pl.ds(s * NRA, NRA)],
                        out_hbm.at[c, pl.ds(s * NRA, NRA)])

    return sk(H2, i_tgt, zeros_nd)


def _precompute_body(src_ref, tgt_ref, ws_ref, wt_ref, b1_ref, a_ref, b_ref):
    a_ref[...] = jnp.dot(src_ref[...], ws_ref[...],
                         preferred_element_type=jnp.float32)
    b_ref[...] = jnp.dot(tgt_ref[...], wt_ref[...],
                         preferred_element_type=jnp.float32) + b1_ref[...]


def _edge_mlp_body(ga_ref, gb_ref, dist_ref, w1d_ref, w2t_ref, b2_ref, out_ref):
    pre = ga_ref[...] + gb_ref[...] + dist_ref[...] * w1d_ref[...]
    h = pre * jax.nn.sigmoid(pre)
    h2 = jnp.dot(h, w2t_ref[...], preferred_element_type=jnp.float32) + b2_ref[...]
    out_ref[...] = h2 * jax.nn.sigmoid(h2)


def _combine_norm_body(tgt_ref, p0_ref, p1_ref, wc1t_t_ref, wc1t_a_ref, bc1_ref,
                       wc2t_ref, bc2_ref, gnw_ref, gnb_ref, gms_ref,
                       tb_col_ref, tb_row_ref, y_ref):
    tgt = tgt_ref[...]
    aggr = p0_ref[...] + p1_ref[...]
    c1 = (jnp.dot(tgt, wc1t_t_ref[...], preferred_element_type=jnp.float32)
          + jnp.dot(aggr, wc1t_a_ref[...],
                    preferred_element_type=jnp.float32)
          + bc1_ref[...])
    c1 = c1 * jax.nn.sigmoid(c1)
    c = jnp.dot(c1, wc2t_ref[...], preferred_element_type=jnp.float32) + bc2_ref[...]
    x = tgt + c

    iota_col = jax.lax.broadcasted_iota(jnp.int32, (N, G), 1)
    onehot = (iota_col == tb_col_ref[...]).astype(jnp.float32)
    iota_row = jax.lax.broadcasted_iota(jnp.int32, (G, N), 0)
    onehot_t = (iota_row == tb_row_ref[...]).astype(jnp.float32)

    counts = jnp.maximum(jnp.sum(onehot_t, axis=1, keepdims=True), 1.0)
    mean = jnp.dot(onehot_t, x, preferred_element_type=jnp.float32) / counts
    out = x - gms_ref[...] * jnp.dot(onehot, mean,
                                     preferred_element_type=jnp.float32)
    var = jnp.dot(onehot_t, out * out,
                  preferred_element_type=jnp.float32) / counts
    rstd = jax.lax.rsqrt(var + 1e-5)
    y_ref[...] = (gnw_ref[...] * out
                  * jnp.dot(onehot, rstd, preferred_element_type=jnp.float32)
                  + gnb_ref[...])


def kernel(source_node, target_node, edge_attr, distance, W1, b1, W2, b2,
           Wc1, bc1, Wc2, bc2, gn_weight, gn_bias, gn_mean_scale,
           edge_index, target_batch):
    del edge_attr
    i_src = edge_index[0]
    i_tgt = edge_index[1]

    W1sT = W1[:, :D].T
    W1tT = W1[:, D:2 * D].T
    w1d = W1[:, 2 * D].reshape(1, H)
    b1r = b1.reshape(1, H)
    W2T = W2.T
    b2r = b2.reshape(1, H)
    Wc1T_t = Wc1[:, :D].T
    Wc1T_a = Wc1[:, D:].T
    bc1r = bc1.reshape(1, H)
    Wc2T = Wc2.T
    bc2r = bc2.reshape(1, H)
    gnw = gn_weight.reshape(1, H)
    gnb = gn_bias.reshape(1, H)
    gms = gn_mean_scale.reshape(1, H)
    tb_col = target_batch.reshape(N, 1)
    tb_row = target_batch.reshape(1, N)

    A, B = pl.pallas_call(
        _precompute_body,
        out_shape=[jax.ShapeDtypeStruct((N, H), jnp.float32),
                   jax.ShapeDtypeStruct((N, H), jnp.float32)],
    )(source_node, target_node, W1sT, W1tT, b1r)

    Ga, Gb = _sc_gather(A, B, i_src, i_tgt)

    grid = E // EBLK
    H2 = pl.pallas_call(
        _edge_mlp_body,
        grid=(grid,),
        in_specs=[
            pl.BlockSpec((EBLK, H), lambda i: (i, 0)),
            pl.BlockSpec((EBLK, H), lambda i: (i, 0)),
            pl.BlockSpec((EBLK, 1), lambda i: (i, 0)),
            pl.BlockSpec((1, H), lambda i: (0, 0)),
            pl.BlockSpec((H, H), lambda i: (0, 0)),
            pl.BlockSpec((1, H), lambda i: (0, 0)),
        ],
        out_specs=pl.BlockSpec((EBLK, H), lambda i: (i, 0)),
        out_shape=jax.ShapeDtypeStruct((E, H), jnp.float32),
    )(Ga, Gb, distance, w1d, W2T, b2r)

    zeros_nd = jnp.zeros((NRA, D), jnp.float32)
    partials = _sc_scatter_add(H2, i_tgt, zeros_nd)

    y = pl.pallas_call(
        _combine_norm_body,
        out_shape=jax.ShapeDtypeStruct((N, H), jnp.float32),
    )(target_node, partials[0, :N], partials[1, :N], Wc1T_t, Wc1T_a, bc1r,
      Wc2T, bc2r, gnw, gnb, gms, tb_col, tb_row)

    return y


# speedup vs baseline: 3.8523x; 1.2442x over previous
import functools

import jax
import jax.numpy as jnp
from jax import lax
from jax.experimental import pallas as pl
from jax.experimental.pallas import tpu as pltpu
from jax.experimental.pallas import tpu_sc as plsc

N = 10000
E = 320000
D = 128
H = 128
G = 16

EBLK = 2000

NC = 2
NS = 16
NW = NC * NS
EW = E // NW
CH = 40
NCH = EW // CH
DEPTH = 5
NRND = NCH // DEPTH

NRA = 632
NPAD = NRA * NS


def _sc_gather(A, B, i_src, i_tgt):
    mesh = plsc.VectorSubcoreMesh(core_axis_name="c", subcore_axis_name="s")

    @functools.partial(
        pl.kernel, mesh=mesh,
        out_type=jax.ShapeDtypeStruct((2, E, D), jnp.float32),
        scratch_types=(
            [pltpu.VMEM((CH,), jnp.int32) for _ in range(DEPTH)]
            + [pltpu.VMEM((CH, D), jnp.float32) for _ in range(DEPTH)]
            + [pltpu.VMEM_SHARED((NPAD, D), jnp.float32)]
            + [pltpu.SemaphoreType.DMA for _ in range(2 * DEPTH)]
        ),
    )
    def gk(a_hbm, b_hbm, is_hbm, it_hbm, out_hbm, *scr):
        idx = scr[:DEPTH]
        rows = scr[DEPTH:2 * DEPTH]
        tab = scr[2 * DEPTH]
        sg = scr[2 * DEPTH + 1:2 * DEPTH + 1 + DEPTH]
        so = scr[2 * DEPTH + 1 + DEPTH:]

        c = lax.axis_index("c")
        s = lax.axis_index("s")
        wid = s * NC + c
        base = wid * EW

        for t, tab_hbm, tidx_hbm in ((0, a_hbm, is_hbm), (1, b_hbm, it_hbm)):
            pltpu.sync_copy(tab_hbm.at[pl.ds(s * NRA, NRA)],
                            tab.at[pl.ds(s * NRA, NRA)])
            plsc.subcore_barrier()

            def start_chunk(k, b):
                off = base + k * CH
                pltpu.sync_copy(tidx_hbm.at[pl.ds(off, CH)], idx[b])
                pltpu.async_copy(tab.at[idx[b]], rows[b], sg[b])

            def finish_chunk(k, b):
                off = base + k * CH
                pltpu.make_async_copy(tab.at[idx[b]], rows[b], sg[b]).wait()
                pltpu.async_copy(rows[b],
                                 out_hbm.at[t, pl.ds(off, CH)], so[b])

            def wait_out(k, b):
                off = base + k * CH
                pltpu.make_async_copy(rows[b],
                                      out_hbm.at[t, pl.ds(off, CH)],
                                      so[b]).wait()

            def round_body(j, carry):
                for b in range(DEPTH):
                    @pl.when(j > 0)
                    def _():
                        wait_out((j - 1) * DEPTH + b, b)
                    start_chunk(j * DEPTH + b, b)
                for b in range(DEPTH):
                    finish_chunk(j * DEPTH + b, b)
                return carry

            lax.fori_loop(0, NRND, round_body, 0)
            for b in range(DEPTH):
                wait_out((NRND - 1) * DEPTH + b, b)
            plsc.subcore_barrier()

    return gk(A, B, i_src, i_tgt)


def _sc_scatter_add(H2, i_tgt, zeros_nd):
    mesh = plsc.VectorSubcoreMesh(core_axis_name="c", subcore_axis_name="s")

    @functools.partial(
        pl.kernel, mesh=mesh,
        out_type=jax.ShapeDtypeStruct((NC, NPAD, D), jnp.float32),
        scratch_types=(
            [pltpu.VMEM((CH,), jnp.int32) for _ in range(DEPTH)]
            + [pltpu.VMEM((CH, D), jnp.float32) for _ in range(DEPTH)]
            + [pltpu.VMEM_SHARED((NPAD, D), jnp.float32)]
            + [pltpu.SemaphoreType.DMA for _ in range(DEPTH)]
        ),
    )
    def sk(h2_hbm, it_hbm, z_hbm, out_hbm, *scr):
        idx = scr[:DEPTH]
        rows = scr[DEPTH:2 * DEPTH]
        acc = scr[2 * DEPTH]
        sr = scr[2 * DEPTH + 1:]

        c = lax.axis_index("c")
        s = lax.axis_index("s")
        pltpu.sync_copy(z_hbm, acc.at[pl.ds(s * NRA, NRA)])

        base = (s * NC + c) * EW

        def start_load(k, b):
            off = base + k * CH
            pltpu.async_copy(h2_hbm.at[pl.ds(off, CH)], rows[b], sr[b])

        def do_scatter(k, b):
            off = base + k * CH
            pltpu.sync_copy(it_hbm.at[pl.ds(off, CH)], idx[b])
            pltpu.make_async_copy(h2_hbm.at[pl.ds(off, CH)],
                                  rows[b], sr[b]).wait()
            pltpu.sync_copy(rows[b], acc.at[idx[b]], add=True)

        for b in range(DEPTH):
            start_load(b, b)
        plsc.subcore_barrier()

        def round_body(j, carry):
            for b in range(DEPTH):
                do_scatter(j * DEPTH + b, b)

                @pl.when(j < NRND - 1)
                def _():
                    start_load((j + 1) * DEPTH + b, b)
            return carry

        lax.fori_loop(0, NRND, round_body, 0)
        plsc.subcore_barrier()
        pltpu.sync_copy(acc.at[pl.ds(s * NRA, NRA)],
                        out_hbm.at[c, pl.ds(s * NRA, NRA)])

    return sk(H2, i_tgt, zeros_nd)


def _precompute_body(src_ref, tgt_ref, ws_ref, wt_ref, b1_ref, a_ref, b_ref):
    a_ref[...] = jnp.dot(src_ref[...], ws_ref[...],
                         preferred_element_type=jnp.float32)
    b_ref[...] = jnp.dot(tgt_ref[...], wt_ref[...],
                         preferred_element_type=jnp.float32) + b1_ref[...]


def _edge_mlp_body(ga_ref, gb_ref, dist_ref, w1d_ref, w2t_ref, b2_ref, out_ref):
    pre = ga_ref[0] + gb_ref[0] + dist_ref[...] * w1d_ref[...]
    h = pre * jax.nn.sigmoid(pre)
    h2 = jnp.dot(h, w2t_ref[...], preferred_element_type=jnp.float32) + b2_ref[...]
    out_ref[...] = h2 * jax.nn.sigmoid(h2)


def _combine_norm_body(tgt_ref, p0_ref, p1_ref, wc1t_t_ref, wc1t_a_ref, bc1_ref,
                       wc2t_ref, bc2_ref, gnw_ref, gnb_ref, gms_ref,
                       tb_col_ref, tb_row_ref, y_ref):
    tgt = tgt_ref[...]
    aggr = p0_ref[...] + p1_ref[...]
    c1 = (jnp.dot(tgt, wc1t_t_ref[...], preferred_element_type=jnp.float32)
          + jnp.dot(aggr, wc1t_a_ref[...],
                    preferred_element_type=jnp.float32)
          + bc1_ref[...])
    c1 = c1 * jax.nn.sigmoid(c1)
    c = jnp.dot(c1, wc2t_ref[...], preferred_element_type=jnp.float32) + bc2_ref[...]
    x = tgt + c

    iota_col = jax.lax.broadcasted_iota(jnp.int32, (N, G), 1)
    onehot = (iota_col == tb_col_ref[...]).astype(jnp.float32)
    iota_row = jax.lax.broadcasted_iota(jnp.int32, (G, N), 0)
    onehot_t = (iota_row == tb_row_ref[...]).astype(jnp.float32)

    counts = jnp.maximum(jnp.sum(onehot_t, axis=1, keepdims=True), 1.0)
    mean = jnp.dot(onehot_t, x, preferred_element_type=jnp.float32) / counts
    out = x - gms_ref[...] * jnp.dot(onehot, mean,
                                     preferred_element_type=jnp.float32)
    var = jnp.dot(onehot_t, out * out,
                  preferred_element_type=jnp.float32) / counts
    rstd = jax.lax.rsqrt(var + 1e-5)
    y_ref[...] = (gnw_ref[...] * out
                  * jnp.dot(onehot, rstd, preferred_element_type=jnp.float32)
                  + gnb_ref[...])


def kernel(source_node, target_node, edge_attr, distance, W1, b1, W2, b2,
           Wc1, bc1, Wc2, bc2, gn_weight, gn_bias, gn_mean_scale,
           edge_index, target_batch):
    del edge_attr

    W1sT = W1[:, :D].T
    W1tT = W1[:, D:2 * D].T
    w1d = W1[:, 2 * D].reshape(1, H)
    b1r = b1.reshape(1, H)
    W2T = W2.T
    b2r = b2.reshape(1, H)
    Wc1T_t = Wc1[:, :D].T
    Wc1T_a = Wc1[:, D:].T
    bc1r = bc1.reshape(1, H)
    Wc2T = Wc2.T
    bc2r = bc2.reshape(1, H)
    gnw = gn_weight.reshape(1, H)
    gnb = gn_bias.reshape(1, H)
    gms = gn_mean_scale.reshape(1, H)
    tb_col = target_batch.reshape(N, 1)
    tb_row = target_batch.reshape(1, N)
    src_pad = jnp.pad(source_node, ((0, NPAD - N), (0, 0)))
    tgt_pad = jnp.pad(target_node, ((0, NPAD - N), (0, 0)))

    A, B = pl.pallas_call(
        _precompute_body,
        out_shape=[jax.ShapeDtypeStruct((NPAD, H), jnp.float32),
                   jax.ShapeDtypeStruct((NPAD, H), jnp.float32)],
    )(src_pad, tgt_pad, W1sT, W1tT, b1r)

    Gab = _sc_gather(A, B, edge_index[0], edge_index[1])

    grid = E // EBLK
    H2 = pl.pallas_call(
        _edge_mlp_body,
        grid=(grid,),
        in_specs=[
            pl.BlockSpec((1, EBLK, H), lambda i: (0, i, 0)),
            pl.BlockSpec((1, EBLK, H), lambda i: (1, i, 0)),
            pl.BlockSpec((EBLK, 1), lambda i: (i, 0)),
            pl.BlockSpec((1, H), lambda i: (0, 0)),
            pl.BlockSpec((H, H), lambda i: (0, 0)),
            pl.BlockSpec((1, H), lambda i: (0, 0)),
        ],
        out_specs=pl.BlockSpec((EBLK, H), lambda i: (i, 0)),
        out_shape=jax.ShapeDtypeStruct((E, H), jnp.float32),
    )(Gab, Gab, distance, w1d, W2T, b2r)

    zeros_nd = jnp.zeros((NRA, D), jnp.float32)
    partials = _sc_scatter_add(H2, edge_index[1], zeros_nd)

    y = pl.pallas_call(
        _combine_norm_body,
        out_shape=jax.ShapeDtypeStruct((N, H), jnp.float32),
    )(target_node, partials[0, :N], partials[1, :N], Wc1T_t, Wc1T_a, bc1r,
      Wc2T, bc2r, gnw, gnb, gms, tb_col, tb_row)

    return y


# speedup vs baseline: 4.4526x; 1.1558x over previous
import functools

import jax
import jax.numpy as jnp
from jax import lax
from jax.experimental import pallas as pl
from jax.experimental.pallas import tpu as pltpu
from jax.experimental.pallas import tpu_sc as plsc

N = 10000
E = 320000
D = 128
H = 128
G = 16

EBLK = 2000

NC = 2
NS = 16
NW = NC * NS
EW = E // NW
CH = 40
DEPTH = 5
NSPLIT = 2
EC = E // NSPLIT

NRA = 632
NPAD = NRA * NS


EW = EC // NW
NCH = EW // CH
NRND = NCH // DEPTH


def _sc_gather(A, B, i_src, i_tgt):
    mesh = plsc.VectorSubcoreMesh(core_axis_name="c", subcore_axis_name="s")

    @functools.partial(
        pl.kernel, mesh=mesh,
        out_type=jax.ShapeDtypeStruct((2, EC, D), jnp.float32),
        scratch_types=(
            [pltpu.VMEM((CH,), jnp.int32) for _ in range(DEPTH)]
            + [pltpu.VMEM((CH, D), jnp.float32) for _ in range(DEPTH)]
            + [pltpu.VMEM_SHARED((NPAD, D), jnp.float32)]
            + [pltpu.SemaphoreType.DMA for _ in range(2 * DEPTH)]
        ),
    )
    def gk(a_hbm, b_hbm, is_hbm, it_hbm, out_hbm, *scr):
        idx = scr[:DEPTH]
        rows = scr[DEPTH:2 * DEPTH]
        tab = scr[2 * DEPTH]
        sg = scr[2 * DEPTH + 1:2 * DEPTH + 1 + DEPTH]
        so = scr[2 * DEPTH + 1 + DEPTH:]

        c = lax.axis_index("c")
        s = lax.axis_index("s")
        wid = s * NC + c
        base = wid * EW

        for t, tab_hbm, tidx_hbm in ((0, a_hbm, is_hbm), (1, b_hbm, it_hbm)):
            pltpu.sync_copy(tab_hbm.at[pl.ds(s * NRA, NRA)],
                            tab.at[pl.ds(s * NRA, NRA)])
            plsc.subcore_barrier()

            def start_chunk(k, b):
                off = base + k * CH
                pltpu.sync_copy(tidx_hbm.at[pl.ds(off, CH)], idx[b])
                pltpu.async_copy(tab.at[idx[b]], rows[b], sg[b])

            def finish_chunk(k, b):
                off = base + k * CH
                pltpu.make_async_copy(tab.at[idx[b]], rows[b], sg[b]).wait()
                pltpu.async_copy(rows[b],
                                 out_hbm.at[t, pl.ds(off, CH)], so[b])

            def wait_out(k, b):
                off = base + k * CH
                pltpu.make_async_copy(rows[b],
                                      out_hbm.at[t, pl.ds(off, CH)],
                                      so[b]).wait()

            def round_body(j, carry):
                for b in range(DEPTH):
                    @pl.when(j > 0)
                    def _():
                        wait_out((j - 1) * DEPTH + b, b)
                    start_chunk(j * DEPTH + b, b)
                for b in range(DEPTH):
                    finish_chunk(j * DEPTH + b, b)
                return carry

            lax.fori_loop(0, NRND, round_body, 0)
            for b in range(DEPTH):
                wait_out((NRND - 1) * DEPTH + b, b)
            plsc.subcore_barrier()

    return gk(A, B, i_src, i_tgt)


def _sc_scatter_add(H2, i_tgt, zeros_nd):
    mesh = plsc.VectorSubcoreMesh(core_axis_name="c", subcore_axis_name="s")

    @functools.partial(
        pl.kernel, mesh=mesh,
        out_type=jax.ShapeDtypeStruct((NC, NPAD, D), jnp.float32),
        scratch_types=(
            [pltpu.VMEM((CH,), jnp.int32) for _ in range(DEPTH)]
            + [pltpu.VMEM((CH, D), jnp.float32) for _ in range(DEPTH)]
            + [pltpu.VMEM_SHARED((NPAD, D), jnp.float32)]
            + [pltpu.SemaphoreType.DMA for _ in range(DEPTH)]
        ),
    )
    def sk(h2_hbm, it_hbm, z_hbm, out_hbm, *scr):
        idx = scr[:DEPTH]
        rows = scr[DEPTH:2 * DEPTH]
        acc = scr[2 * DEPTH]
        sr = scr[2 * DEPTH + 1:]

        c = lax.axis_index("c")
        s = lax.axis_index("s")
        pltpu.sync_copy(z_hbm, acc.at[pl.ds(s * NRA, NRA)])

        base = (s * NC + c) * EW

        def start_load(k, b):
            off = base + k * CH
            pltpu.async_copy(h2_hbm.at[pl.ds(off, CH)], rows[b], sr[b])

        def do_scatter(k, b):
            off = base + k * CH
            pltpu.sync_copy(it_hbm.at[pl.ds(off, CH)], idx[b])
            pltpu.make_async_copy(h2_hbm.at[pl.ds(off, CH)],
                                  rows[b], sr[b]).wait()
            pltpu.sync_copy(rows[b], acc.at[idx[b]], add=True)

        for b in range(DEPTH):
            start_load(b, b)
        plsc.subcore_barrier()

        def round_body(j, carry):
            for b in range(DEPTH):
                do_scatter(j * DEPTH + b, b)

                @pl.when(j < NRND - 1)
                def _():
                    start_load((j + 1) * DEPTH + b, b)
            return carry

        lax.fori_loop(0, NRND, round_body, 0)
        plsc.subcore_barrier()
        pltpu.sync_copy(acc.at[pl.ds(s * NRA, NRA)],
                        out_hbm.at[c, pl.ds(s * NRA, NRA)])

    return sk(H2, i_tgt, zeros_nd)


def _precompute_body(src_ref, tgt_ref, ws_ref, wt_ref, b1_ref, a_ref, b_ref):
    a_ref[...] = jnp.dot(src_ref[...], ws_ref[...],
                         preferred_element_type=jnp.float32)
    b_ref[...] = jnp.dot(tgt_ref[...], wt_ref[...],
                         preferred_element_type=jnp.float32) + b1_ref[...]


def _edge_mlp_body(ga_ref, gb_ref, dist_ref, w1d_ref, w2t_ref, b2_ref, out_ref):
    pre = ga_ref[0] + gb_ref[0] + dist_ref[...] * w1d_ref[...]
    h = pre * jax.nn.sigmoid(pre)
    h2 = jnp.dot(h, w2t_ref[...], preferred_element_type=jnp.float32) + b2_ref[...]
    out_ref[...] = h2 * jax.nn.sigmoid(h2)


def _combine_norm_body(tgt_ref, p0_ref, p1_ref, p2_ref, p3_ref,
                       wc1t_t_ref, wc1t_a_ref, bc1_ref,
                       wc2t_ref, bc2_ref, gnw_ref, gnb_ref, gms_ref,
                       tb_col_ref, tb_row_ref, y_ref):
    tgt = tgt_ref[...]
    aggr = (p0_ref[...] + p1_ref[...]) + (p2_ref[...] + p3_ref[...])
    c1 = (jnp.dot(tgt, wc1t_t_ref[...], preferred_element_type=jnp.float32)
          + jnp.dot(aggr, wc1t_a_ref[...],
                    preferred_element_type=jnp.float32)
          + bc1_ref[...])
    c1 = c1 * jax.nn.sigmoid(c1)
    c = jnp.dot(c1, wc2t_ref[...], preferred_element_type=jnp.float32) + bc2_ref[...]
    x = tgt + c

    iota_col = jax.lax.broadcasted_iota(jnp.int32, (N, G), 1)
    onehot = (iota_col == tb_col_ref[...]).astype(jnp.float32)
    iota_row = jax.lax.broadcasted_iota(jnp.int32, (G, N), 0)
    onehot_t = (iota_row == tb_row_ref[...]).astype(jnp.float32)

    counts = jnp.maximum(jnp.sum(onehot_t, axis=1, keepdims=True), 1.0)
    mean = jnp.dot(onehot_t, x, preferred_element_type=jnp.float32) / counts
    out = x - gms_ref[...] * jnp.dot(onehot, mean,
                                     preferred_element_type=jnp.float32)
    var = jnp.dot(onehot_t, out * out,
                  preferred_element_type=jnp.float32) / counts
    rstd = jax.lax.rsqrt(var + 1e-5)
    y_ref[...] = (gnw_ref[...] * out
                  * jnp.dot(onehot, rstd, preferred_element_type=jnp.float32)
                  + gnb_ref[...])


def kernel(source_node, target_node, edge_attr, distance, W1, b1, W2, b2,
           Wc1, bc1, Wc2, bc2, gn_weight, gn_bias, gn_mean_scale,
           edge_index, target_batch):
    del edge_attr

    W1sT = W1[:, :D].T
    W1tT = W1[:, D:2 * D].T
    w1d = W1[:, 2 * D].reshape(1, H)
    b1r = b1.reshape(1, H)
    W2T = W2.T
    b2r = b2.reshape(1, H)
    Wc1T_t = Wc1[:, :D].T
    Wc1T_a = Wc1[:, D:].T
    bc1r = bc1.reshape(1, H)
    Wc2T = Wc2.T
    bc2r = bc2.reshape(1, H)
    gnw = gn_weight.reshape(1, H)
    gnb = gn_bias.reshape(1, H)
    gms = gn_mean_scale.reshape(1, H)
    tb_col = target_batch.reshape(N, 1)
    tb_row = target_batch.reshape(1, N)
    src_pad = jnp.pad(source_node, ((0, NPAD - N), (0, 0)))
    tgt_pad = jnp.pad(target_node, ((0, NPAD - N), (0, 0)))

    A, B = pl.pallas_call(
        _precompute_body,
        out_shape=[jax.ShapeDtypeStruct((NPAD, H), jnp.float32),
                   jax.ShapeDtypeStruct((NPAD, H), jnp.float32)],
    )(src_pad, tgt_pad, W1sT, W1tT, b1r)

    zeros_nd = jnp.zeros((NRA, D), jnp.float32)
    grid = EC // EBLK
    partials = []
    for ci in range(NSPLIT):
        lo = ci * EC
        isrc = lax.dynamic_slice_in_dim(edge_index[0], lo, EC)
        itgt = lax.dynamic_slice_in_dim(edge_index[1], lo, EC)
        dist = lax.dynamic_slice_in_dim(distance, lo, EC)

        Gab = _sc_gather(A, B, isrc, itgt)

        H2 = pl.pallas_call(
            _edge_mlp_body,
            grid=(grid,),
            in_specs=[
                pl.BlockSpec((1, EBLK, H), lambda i: (0, i, 0)),
                pl.BlockSpec((1, EBLK, H), lambda i: (1, i, 0)),
                pl.BlockSpec((EBLK, 1), lambda i: (i, 0)),
                pl.BlockSpec((1, H), lambda i: (0, 0)),
                pl.BlockSpec((H, H), lambda i: (0, 0)),
                pl.BlockSpec((1, H), lambda i: (0, 0)),
            ],
            out_specs=pl.BlockSpec((EBLK, H), lambda i: (i, 0)),
            out_shape=jax.ShapeDtypeStruct((EC, H), jnp.float32),
        )(Gab, Gab, dist, w1d, W2T, b2r)

        partials.append(_sc_scatter_add(H2, itgt, zeros_nd))

    psum = [p[c, :N] for p in partials for c in range(NC)]
    y = pl.pallas_call(
        _combine_norm_body,
        out_shape=jax.ShapeDtypeStruct((N, H), jnp.float32),
    )(target_node, psum[0], psum[1], psum[2], psum[3], Wc1T_t, Wc1T_a, bc1r,
      Wc2T, bc2r, gnw, gnb, gms, tb_col, tb_row)

    return y
